# baseline (speedup 1.0000x reference)
import functools

import jax
import jax.numpy as jnp
from jax import lax
from jax.experimental import pallas as pl
from jax.experimental.pallas import tpu as pltpu
from jax.experimental.pallas import tpu_sc as plsc

N = 10000
NPAD = 10112
RPT = 632
DIN = 128
H1, DH1 = 8, 8
F1 = H1 * DH1
NC_CLS = 40

E_RAW = 320000
E = E_RAW + N
BLK = 128
CPT = 10368
NBLK = CPT // BLK
NWORK = 32
EPAD = CPT * NWORK

RW1 = 80
TW1 = 48
RW2 = 48

_f32 = jnp.float32


def _lrelu(x):
  return jnp.maximum(x, 0.2 * x)


def _perm(v, idx):
  return lax.gather(
      v, idx[:, None],
      lax.GatherDimensionNumbers(
          offset_dims=(), collapsed_slice_dims=(0,), start_index_map=(0,)),
      slice_sizes=(1,), mode=lax.GatherScatterMode.PROMISE_IN_BOUNDS)


def _tc_stage1(xp, w1f, a_s, a_d):
  def body(x_ref, w_ref, as_ref, ad_ref, tsrc_ref, ted_ref):
    x = x_ref[...]
    wh = jnp.dot(x, w_ref[...], preferred_element_type=_f32)
    es = jnp.dot(wh, as_ref[...], preferred_element_type=_f32)
    ed = jnp.dot(wh, ad_ref[...], preferred_element_type=_f32)
    esm = jnp.max(es, axis=0, keepdims=True)
    b = _lrelu(esm + ed)
    tsrc_ref[...] = jnp.concatenate(
        [wh, es, jnp.zeros((NPAD, 8), _f32)], axis=1)
    ted_ref[...] = jnp.concatenate([ed, b], axis=1)

  whes, ted = pl.pallas_call(
      body,
      out_shape=[
          jax.ShapeDtypeStruct((NPAD, F1 + 16), _f32),
          jax.ShapeDtypeStruct((NPAD, 16), _f32),
      ],
  )(xp, w1f, a_s, a_d)
  whr = whes[:, :F1].astype(jnp.bfloat16).reshape(NPAD, 2, 2, 16)
  iv = jnp.stack([whr[:, :, 0, :], whr[:, :, 1, :]], axis=-1)
  packed = jax.lax.bitcast_convert_type(iv, _f32).reshape(NPAD, 32)
  tsrc = jnp.concatenate([packed, whes[:, F1:]], axis=1)
  return tsrc, ted


def _tc_stage2(acc1, w2f, a2s, a2d):
  def body(acc_ref, w_ref, as_ref, ad_ref, tsrc_ref, ted_ref):
    sa = acc_ref[0] + acc_ref[1]
    num = sa[:, :F1]
    den = sa[:, F1:F1 + H1]
    rep = (jnp.arange(H1, dtype=jnp.int32)[:, None]
           == (jnp.arange(F1, dtype=jnp.int32)[None, :] // DH1)).astype(_f32)
    denr = jnp.dot(den, rep, preferred_element_type=_f32)
    x2 = num / (denr + 1e-16)
    x2 = jnp.where(x2 > 0, x2, jnp.exp(jnp.minimum(x2, 0.0)) - 1.0)
    wh2 = jnp.dot(x2, w_ref[...], preferred_element_type=_f32)
    es2 = jnp.dot(wh2, as_ref[...], preferred_element_type=_f32)
    ed2 = jnp.dot(wh2, ad_ref[...], preferred_element_type=_f32)
    esm2 = jnp.max(es2)
    b2 = _lrelu(esm2 + ed2)
    tsrc_ref[...] = jnp.concatenate(
        [wh2, es2, jnp.zeros((NPAD, 7), _f32)], axis=1)
    ted_ref[...] = jnp.concatenate(
        [ed2, b2, jnp.zeros((NPAD, 14), _f32)], axis=1)

  return pl.pallas_call(
      body,
      out_shape=[
          jax.ShapeDtypeStruct((NPAD, RW2), _f32),
          jax.ShapeDtypeStruct((NPAD, 16), _f32),
      ],
  )(acc1, w2f, a2s, a2d)


def _tc_stage3(acc2):
  def body(acc_ref, out_ref):
    sb = acc_ref[0] + acc_ref[1]
    logits = sb[:, :NC_CLS] / (sb[:, NC_CLS:NC_CLS + 1] + 1e-16)
    mx = jnp.max(logits, axis=1, keepdims=True)
    lse = jnp.log(jnp.sum(jnp.exp(logits - mx), axis=1, keepdims=True)) + mx
    out_ref[...] = (logits - lse)[:N]

  return pl.pallas_call(
      body,
      out_shape=jax.ShapeDtypeStruct((N, NC_CLS), _f32),
  )(acc2)


def _acc_chunks():
  out, off = [], 0
  while off < RPT:
    n = min(BLK, RPT - off)
    out.append((off, n))
    off += n
  return out


def _make_edge_pass(tw, rw, nheads):
  mesh = plsc.VectorSubcoreMesh(core_axis_name="c", subcore_axis_name="s")
  NBUF = 3
  packed = tw != rw

  def body(tsrc, ted, sidx_h, didx_h, out,
           sidx, didx, bufs, tbufs, msgs, acc, *sems):
    sem_gs = sems[0:NBUF]
    sem_gt = sems[NBUF:2 * NBUF]
    sem_sc = sems[2 * NBUF:3 * NBUF]
    c = lax.axis_index("c")
    s = lax.axis_index("s")
    wid = s * 2 + c

    iota = lax.iota(jnp.int32, 16)

    base = wid * NBLK
    pltpu.sync_copy(sidx_h.at[pl.ds(base, NBLK)], sidx)
    pltpu.sync_copy(didx_h.at[pl.ds(base, NBLK)], didx)

    def zrow(i, _):
      for q in range(rw // 16):
        msgs[0, i, pl.ds(q * 16, 16)] = jnp.zeros((16,), _f32)
      return _
    lax.fori_loop(0, BLK, zrow, None)
    row0 = s * RPT
    for off, n in _acc_chunks():
      pltpu.sync_copy(msgs.at[0, pl.ds(0, n)], acc.at[pl.ds(row0 + off, n)])
    plsc.subcore_barrier()

    if nheads == 8:
      def edge_one(b, i):
        t = tbufs[b, i, pl.ds(0, 16)]
        u = bufs[b, i, pl.ds(32, 16)]
        z = u + t
        bb = _perm(t, 8 + (iota & 7))
        ex = jnp.exp(_lrelu(z) - bb)
        msgs[b, i, pl.ds(F1, 16)] = jnp.where(iota < 8, ex, 0.0)
        for g in range(2):
          w = bufs[b, i, pl.ds(g * 16, 16)]
          lo, hi = plsc.unpack(plsc.bitcast(w, jnp.bfloat16),
                               format=plsc.PackFormat.INTERLEAVED,
                               preferred_element_type=_f32)
          e0 = _perm(ex, 4 * g + (iota >> 3))
          e1 = _perm(ex, 4 * g + 2 + (iota >> 3))
          msgs[b, i, pl.ds(32 * g, 16)] = lo * e0
          msgs[b, i, pl.ds(32 * g + 16, 16)] = hi * e1
    else:
      def edge_one(b, i):
        t = tbufs[b, i, pl.ds(0, 16)]
        w2 = bufs[b, i, pl.ds(32, 16)]
        ed = _perm(t, iota * 0)
        bb = _perm(t, iota * 0 + 1)
        es = _perm(w2, iota * 0 + 8)
        ex = jnp.exp(_lrelu(es + ed) - bb)
        w2m = jnp.where(iota == 8, 1.0, w2)
        msgs[b, i, pl.ds(32, 16)] = w2m * ex
        w0 = bufs[b, i, pl.ds(0, 16)]
        msgs[b, i, pl.ds(0, 16)] = w0 * ex
        w1 = bufs[b, i, pl.ds(16, 16)]
        msgs[b, i, pl.ds(16, 16)] = w1 * ex

    def start_gather(j, b):
      pltpu.make_async_copy(
          tsrc.at[sidx.at[j]], bufs.at[b], sem_gs[b]).start()
      pltpu.make_async_copy(
          ted.at[didx.at[j]], tbufs.at[b], sem_gt[b]).start()

    def wait_gather(j, b):
      pltpu.make_async_copy(
          tsrc.at[sidx.at[j]], bufs.at[b], sem_gs[b]).wait()
      pltpu.make_async_copy(
          ted.at[didx.at[j]], tbufs.at[b], sem_gt[b]).wait()

    def start_scatter(j, b):
      pltpu.async_copy(msgs.at[b], acc.at[didx.at[j]], sem_sc[b], add=True)

    def wait_scatter(j, b):
      pltpu.make_async_copy(
          msgs.at[b], acc.at[didx.at[j]], sem_sc[b]).wait()

    start_gather(0, 0)
    start_gather(1, 1)

    def ring_body(g, _):
      j0 = g * NBUF
      for b in range(NBUF):
        j = j0 + b
        wait_gather(j, b)

        @plsc.parallel_loop(0, BLK, 1, unroll=8)
        def _compute(i, b=b):
          edge_one(b, i)
        start_scatter(j, b)
        bn = (b + 2) % NBUF
        jn = j + 2

        @pl.when(jn < NBLK)
        def _issue():
          @pl.when(j >= 1)
          def _drain():
            wait_scatter(j - 1, bn)
          start_gather(jn, bn)
      return _
    lax.fori_loop(0, NBLK // NBUF, ring_body, None)

    for j in range(NBLK - NBUF, NBLK):
      wait_scatter(j, j % NBUF)
    plsc.subcore_barrier()
    for off, n in _acc_chunks():
      pltpu.sync_copy(acc.at[pl.ds(row0 + off, n)],
                      out.at[c, pl.ds(row0 + off, n)])

  return functools.partial(
      pl.kernel,
      out_type=jax.ShapeDtypeStruct((2, NPAD, rw), _f32),
      mesh=mesh,
      compiler_params=pltpu.CompilerParams(use_tc_tiling_on_sc=False,
                                           needs_layout_passes=False),
      scratch_types=[
          pltpu.VMEM((NBLK, BLK), jnp.int32),
          pltpu.VMEM((NBLK, BLK), jnp.int32),
          pltpu.VMEM((NBUF, BLK, tw), _f32),
          pltpu.VMEM((NBUF, BLK, 16), _f32),
          pltpu.VMEM((NBUF, BLK, rw), _f32),
          pltpu.VMEM_SHARED((NPAD, rw), _f32),
      ] + [pltpu.SemaphoreType.DMA] * (3 * NBUF),
  )(body)


_edge_pass_l1 = _make_edge_pass(TW1, RW1, 8)
_edge_pass_l2 = _make_edge_pass(RW2, RW2, 1)


def kernel(input_matrix, adjacency_coo_matrix, W1, a_src1, a_dst1,
           W2, a_src2, a_dst2):
  xp = jnp.zeros((NPAD, DIN), _f32).at[:N].set(input_matrix)
  w1f = jnp.transpose(W1, (1, 0, 2)).reshape(DIN, F1)
  hh = jnp.repeat(jnp.arange(H1, dtype=jnp.int32), DH1)
  a_s = jnp.zeros((F1, H1), _f32).at[jnp.arange(F1), hh].set(a_src1.reshape(F1))
  a_d = jnp.zeros((F1, H1), _f32).at[jnp.arange(F1), hh].set(a_dst1.reshape(F1))
  w2f = W2.reshape(F1, NC_CLS)
  a2s = a_src2.reshape(NC_CLS, 1)
  a2d = a_dst2.reshape(NC_CLS, 1)

  adj = adjacency_coo_matrix.astype(jnp.int32)
  ar = jnp.arange(N, dtype=jnp.int32)
  fill = N + jnp.arange(EPAD - E, dtype=jnp.int32) % (NPAD - N)
  src = jnp.concatenate([adj[0], ar, fill]).reshape(EPAD // BLK, BLK)
  dst = jnp.concatenate([adj[1], ar, fill]).reshape(EPAD // BLK, BLK)

  tsrc1, ted1 = _tc_stage1(xp, w1f, a_s, a_d)
  acc1 = _edge_pass_l1(tsrc1, ted1, src, dst)
  tsrc2, ted2 = _tc_stage2(acc1, w2f, a2s, a2d)
  acc2 = _edge_pass_l2(tsrc2, ted2, src, dst)
  return _tc_stage3(acc2)


# speedup vs baseline: 1.1047x; 1.1047x over previous
import functools

import jax
import jax.numpy as jnp
from jax import lax
from jax.experimental import pallas as pl
from jax.experimental.pallas import tpu as pltpu
from jax.experimental.pallas import tpu_sc as plsc

N = 10000
NPAD = 10112
RPT = 632
DIN = 128
H1, DH1 = 8, 8
F1 = H1 * DH1
NC_CLS = 40

E_RAW = 320000
E = E_RAW + N
BLK = 128
CPT = 10368
NBLK = CPT // BLK
NWORK = 32
EPAD = CPT * NWORK

RW1 = 80
TW1 = 48
RW2 = 48

_f32 = jnp.float32


def _lrelu(x):
  return jnp.maximum(x, 0.2 * x)


def _perm(v, idx):
  return lax.gather(
      v, idx[:, None],
      lax.GatherDimensionNumbers(
          offset_dims=(), collapsed_slice_dims=(0,), start_index_map=(0,)),
      slice_sizes=(1,), mode=lax.GatherScatterMode.PROMISE_IN_BOUNDS)


def _tc_stage1(xp, w1f, a_s, a_d):
  def body(x_ref, w_ref, as_ref, ad_ref, tsrc_ref, ted_ref):
    x = x_ref[...]
    wh = jnp.dot(x, w_ref[...], preferred_element_type=_f32)
    es = jnp.dot(wh, as_ref[...], preferred_element_type=_f32)
    ed = jnp.dot(wh, ad_ref[...], preferred_element_type=_f32)
    esm = jnp.max(es, axis=0, keepdims=True)
    b = _lrelu(esm + ed)
    whq = wh.astype(jnp.bfloat16).astype(_f32)
    whu = jax.lax.bitcast_convert_type(whq, jnp.uint32)
    groups = []
    for g in range(2):
      lo = whu[:, 32 * g:32 * g + 16]
      hi = whu[:, 32 * g + 16:32 * g + 32]
      groups.append(hi | (lo >> 16))
    packed = jax.lax.bitcast_convert_type(
        jnp.concatenate(groups, axis=1), _f32)
    tsrc_ref[...] = jnp.concatenate(
        [packed, es, jnp.zeros((NPAD, 8), _f32)], axis=1)
    ted_ref[...] = jnp.concatenate([ed, b], axis=1)

  return pl.pallas_call(
      body,
      out_shape=[
          jax.ShapeDtypeStruct((NPAD, TW1), _f32),
          jax.ShapeDtypeStruct((NPAD, 16), _f32),
      ],
  )(xp, w1f, a_s, a_d)


def _tc_stage2(acc1, w2f, a2s, a2d):
  def body(acc_ref, w_ref, as_ref, ad_ref, tsrc_ref, ted_ref):
    sa = acc_ref[0] + acc_ref[1]
    num = sa[:, :F1]
    den = sa[:, F1:F1 + H1]
    rep = (jnp.arange(H1, dtype=jnp.int32)[:, None]
           == (jnp.arange(F1, dtype=jnp.int32)[None, :] // DH1)).astype(_f32)
    denr = jnp.dot(den, rep, preferred_element_type=_f32)
    x2 = num / (denr + 1e-16)
    x2 = jnp.where(x2 > 0, x2, jnp.exp(jnp.minimum(x2, 0.0)) - 1.0)
    wh2 = jnp.dot(x2, w_ref[...], preferred_element_type=_f32)
    es2 = jnp.dot(wh2, as_ref[...], preferred_element_type=_f32)
    ed2 = jnp.dot(wh2, ad_ref[...], preferred_element_type=_f32)
    esm2 = jnp.max(es2)
    b2 = _lrelu(esm2 + ed2)
    tsrc_ref[...] = jnp.concatenate(
        [wh2, es2, jnp.zeros((NPAD, 7), _f32)], axis=1)
    ted_ref[...] = jnp.concatenate(
        [ed2, b2, jnp.zeros((NPAD, 14), _f32)], axis=1)

  return pl.pallas_call(
      body,
      out_shape=[
          jax.ShapeDtypeStruct((NPAD, RW2), _f32),
          jax.ShapeDtypeStruct((NPAD, 16), _f32),
      ],
  )(acc1, w2f, a2s, a2d)


def _tc_stage3(acc2):
  def body(acc_ref, out_ref):
    sb = acc_ref[0] + acc_ref[1]
    logits = sb[:, :NC_CLS] / (sb[:, NC_CLS:NC_CLS + 1] + 1e-16)
    mx = jnp.max(logits, axis=1, keepdims=True)
    lse = jnp.log(jnp.sum(jnp.exp(logits - mx), axis=1, keepdims=True)) + mx
    out_ref[...] = (logits - lse)[:N]

  return pl.pallas_call(
      body,
      out_shape=jax.ShapeDtypeStruct((N, NC_CLS), _f32),
  )(acc2)


def _acc_chunks():
  out, off = [], 0
  while off < RPT:
    n = min(BLK, RPT - off)
    out.append((off, n))
    off += n
  return out


def _make_edge_pass(tw, rw, nheads):
  mesh = plsc.VectorSubcoreMesh(core_axis_name="c", subcore_axis_name="s")
  NBUF = 3
  packed = tw != rw

  def body(tsrc, ted, sidx_h, didx_h, out,
           sidx, didx, bufs, tbufs, msgs, acc, *sems):
    sem_gs = sems[0:NBUF]
    sem_gt = sems[NBUF:2 * NBUF]
    sem_sc = sems[2 * NBUF:3 * NBUF]
    c = lax.axis_index("c")
    s = lax.axis_index("s")
    wid = s * 2 + c

    iota = lax.iota(jnp.int32, 16)

    base = wid * NBLK
    pltpu.sync_copy(sidx_h.at[pl.ds(base, NBLK)], sidx)
    pltpu.sync_copy(didx_h.at[pl.ds(base, NBLK)], didx)

    def zrow(i, _):
      for q in range(rw // 16):
        msgs[0, i, pl.ds(q * 16, 16)] = jnp.zeros((16,), _f32)
      return _
    lax.fori_loop(0, BLK, zrow, None)
    row0 = s * RPT
    for off, n in _acc_chunks():
      pltpu.sync_copy(msgs.at[0, pl.ds(0, n)], acc.at[pl.ds(row0 + off, n)])
    plsc.subcore_barrier()

    if nheads == 8:
      def edge_one(b, i):
        t = tbufs[b, i, pl.ds(0, 16)]
        u = bufs[b, i, pl.ds(32, 16)]
        z = u + t
        bb = _perm(t, 8 + (iota & 7))
        ex = jnp.exp(_lrelu(z) - bb)
        msgs[b, i, pl.ds(F1, 16)] = jnp.where(iota < 8, ex, 0.0)
        for g in range(2):
          w = bufs[b, i, pl.ds(g * 16, 16)]
          lo, hi = plsc.unpack(plsc.bitcast(w, jnp.bfloat16),
                               format=plsc.PackFormat.INTERLEAVED,
                               preferred_element_type=_f32)
          e0 = _perm(ex, 4 * g + (iota >> 3))
          e1 = _perm(ex, 4 * g + 2 + (iota >> 3))
          msgs[b, i, pl.ds(32 * g, 16)] = lo * e0
          msgs[b, i, pl.ds(32 * g + 16, 16)] = hi * e1
    else:
      def edge_one(b, i):
        t = tbufs[b, i, pl.ds(0, 16)]
        w2 = bufs[b, i, pl.ds(32, 16)]
        ed = _perm(t, iota * 0)
        bb = _perm(t, iota * 0 + 1)
        es = _perm(w2, iota * 0 + 8)
        ex = jnp.exp(_lrelu(es + ed) - bb)
        w2m = jnp.where(iota == 8, 1.0, w2)
        msgs[b, i, pl.ds(32, 16)] = w2m * ex
        w0 = bufs[b, i, pl.ds(0, 16)]
        msgs[b, i, pl.ds(0, 16)] = w0 * ex
        w1 = bufs[b, i, pl.ds(16, 16)]
        msgs[b, i, pl.ds(16, 16)] = w1 * ex

    def start_gather(j, b):
      pltpu.make_async_copy(
          tsrc.at[sidx.at[j]], bufs.at[b], sem_gs[b]).start()
      pltpu.make_async_copy(
          ted.at[didx.at[j]], tbufs.at[b], sem_gt[b]).start()

    def wait_gather(j, b):
      pltpu.make_async_copy(
          tsrc.at[sidx.at[j]], bufs.at[b], sem_gs[b]).wait()
      pltpu.make_async_copy(
          ted.at[didx.at[j]], tbufs.at[b], sem_gt[b]).wait()

    def start_scatter(j, b):
      pltpu.async_copy(msgs.at[b], acc.at[didx.at[j]], sem_sc[b], add=True)

    def wait_scatter(j, b):
      pltpu.make_async_copy(
          msgs.at[b], acc.at[didx.at[j]], sem_sc[b]).wait()

    start_gather(0, 0)
    start_gather(1, 1)

    def ring_body(g, _):
      j0 = g * NBUF
      for b in range(NBUF):
        j = j0 + b
        wait_gather(j, b)

        @plsc.parallel_loop(0, BLK, 1, unroll=8)
        def _compute(i, b=b):
          edge_one(b, i)
        start_scatter(j, b)
        bn = (b + 2) % NBUF
        jn = j + 2

        @pl.when(jn < NBLK)
        def _issue():
          @pl.when(j >= 1)
          def _drain():
            wait_scatter(j - 1, bn)
          start_gather(jn, bn)
      return _
    lax.fori_loop(0, NBLK // NBUF, ring_body, None)

    for j in range(NBLK - NBUF, NBLK):
      wait_scatter(j, j % NBUF)
    plsc.subcore_barrier()
    for off, n in _acc_chunks():
      pltpu.sync_copy(acc.at[pl.ds(row0 + off, n)],
                      out.at[c, pl.ds(row0 + off, n)])

  return functools.partial(
      pl.kernel,
      out_type=jax.ShapeDtypeStruct((2, NPAD, rw), _f32),
      mesh=mesh,
      compiler_params=pltpu.CompilerParams(use_tc_tiling_on_sc=False,
                                           needs_layout_passes=False),
      scratch_types=[
          pltpu.VMEM((NBLK, BLK), jnp.int32),
          pltpu.VMEM((NBLK, BLK), jnp.int32),
          pltpu.VMEM((NBUF, BLK, tw), _f32),
          pltpu.VMEM((NBUF, BLK, 16), _f32),
          pltpu.VMEM((NBUF, BLK, rw), _f32),
          pltpu.VMEM_SHARED((NPAD, rw), _f32),
      ] + [pltpu.SemaphoreType.DMA] * (3 * NBUF),
  )(body)


_edge_pass_l1 = _make_edge_pass(TW1, RW1, 8)
_edge_pass_l2 = _make_edge_pass(RW2, RW2, 1)


def kernel(input_matrix, adjacency_coo_matrix, W1, a_src1, a_dst1,
           W2, a_src2, a_dst2):
  xp = jnp.zeros((NPAD, DIN), _f32).at[:N].set(input_matrix)
  w1f = jnp.transpose(W1, (1, 0, 2)).reshape(DIN, F1)
  hh = jnp.repeat(jnp.arange(H1, dtype=jnp.int32), DH1)
  a_s = jnp.zeros((F1, H1), _f32).at[jnp.arange(F1), hh].set(a_src1.reshape(F1))
  a_d = jnp.zeros((F1, H1), _f32).at[jnp.arange(F1), hh].set(a_dst1.reshape(F1))
  w2f = W2.reshape(F1, NC_CLS)
  a2s = a_src2.reshape(NC_CLS, 1)
  a2d = a_dst2.reshape(NC_CLS, 1)

  adj = adjacency_coo_matrix.astype(jnp.int32)
  ar = jnp.arange(N, dtype=jnp.int32)
  fill = N + jnp.arange(EPAD - E, dtype=jnp.int32) % (NPAD - N)
  src = jnp.concatenate([adj[0], ar, fill]).reshape(EPAD // BLK, BLK)
  dst = jnp.concatenate([adj[1], ar, fill]).reshape(EPAD // BLK, BLK)

  tsrc1, ted1 = _tc_stage1(xp, w1f, a_s, a_d)
  acc1 = _edge_pass_l1(tsrc1, ted1, src, dst)
  tsrc2, ted2 = _tc_stage2(acc1, w2f, a2s, a2d)
  acc2 = _edge_pass_l2(tsrc2, ted2, src, dst)
  return _tc_stage3(acc2)


# speedup vs baseline: 1.1146x; 1.0090x over previous
import functools

import jax
import jax.numpy as jnp
from jax import lax
from jax.experimental import pallas as pl
from jax.experimental.pallas import tpu as pltpu
from jax.experimental.pallas import tpu_sc as plsc

N = 10000
NPAD = 10112
RPT = 632
DIN = 128
H1, DH1 = 8, 8
F1 = H1 * DH1
NC_CLS = 40

E_RAW = 320000
E = E_RAW + N
BLK = 128
CPT = 10368
NBLK = CPT // BLK
NWORK = 32
EPAD = CPT * NWORK

RW1 = 80
TW1 = 48
RW2 = 48
TW2 = 32

_f32 = jnp.float32


def _lrelu(x):
  return jnp.maximum(x, 0.2 * x)


def _perm(v, idx):
  return lax.gather(
      v, idx[:, None],
      lax.GatherDimensionNumbers(
          offset_dims=(), collapsed_slice_dims=(0,), start_index_map=(0,)),
      slice_sizes=(1,), mode=lax.GatherScatterMode.PROMISE_IN_BOUNDS)


def _tc_stage1(xp, w1f, a_s, a_d):
  def body(x_ref, w_ref, as_ref, ad_ref, tsrc_ref, ted_ref):
    x = x_ref[...]
    wh = jnp.dot(x, w_ref[...], preferred_element_type=_f32)
    es = jnp.dot(wh, as_ref[...], preferred_element_type=_f32)
    ed = jnp.dot(wh, ad_ref[...], preferred_element_type=_f32)
    esm = jnp.max(es, axis=0, keepdims=True)
    b = _lrelu(esm + ed)
    whq = wh.astype(jnp.bfloat16).astype(_f32)
    whu = jax.lax.bitcast_convert_type(whq, jnp.uint32)
    groups = []
    for g in range(2):
      lo = whu[:, 32 * g:32 * g + 16]
      hi = whu[:, 32 * g + 16:32 * g + 32]
      groups.append(hi | (lo >> 16))
    packed = jax.lax.bitcast_convert_type(
        jnp.concatenate(groups, axis=1), _f32)
    tsrc_ref[...] = jnp.concatenate(
        [packed, es, jnp.zeros((NPAD, 8), _f32)], axis=1)
    ted_ref[...] = jnp.concatenate([ed, b], axis=1)

  return pl.pallas_call(
      body,
      out_shape=[
          jax.ShapeDtypeStruct((NPAD, TW1), _f32),
          jax.ShapeDtypeStruct((NPAD, 16), _f32),
      ],
  )(xp, w1f, a_s, a_d)


def _tc_stage2(acc1, w2f, a2s, a2d):
  def body(acc_ref, w_ref, as_ref, ad_ref, tsrc_ref, ted_ref):
    sa = acc_ref[0] + acc_ref[1]
    num = sa[:, :F1]
    den = sa[:, F1:F1 + H1]
    rep = (jnp.arange(H1, dtype=jnp.int32)[:, None]
           == (jnp.arange(F1, dtype=jnp.int32)[None, :] // DH1)).astype(_f32)
    denr = jnp.dot(den, rep, preferred_element_type=_f32)
    x2 = num / (denr + 1e-16)
    x2 = jnp.where(x2 > 0, x2, jnp.exp(jnp.minimum(x2, 0.0)) - 1.0)
    wh2 = jnp.dot(x2, w_ref[...], preferred_element_type=_f32)
    es2 = jnp.dot(wh2, as_ref[...], preferred_element_type=_f32)
    ed2 = jnp.dot(wh2, ad_ref[...], preferred_element_type=_f32)
    esm2 = jnp.max(es2)
    b2 = _lrelu(esm2 + ed2)
    whq = wh2[:, :32].astype(jnp.bfloat16).astype(_f32)
    whu = jax.lax.bitcast_convert_type(whq, jnp.uint32)
    packed = jax.lax.bitcast_convert_type(
        whu[:, 16:32] | (whu[:, 0:16] >> 16), _f32)
    tsrc_ref[...] = jnp.concatenate(
        [packed, wh2[:, 32:40], es2, jnp.zeros((NPAD, 7), _f32)], axis=1)
    ted_ref[...] = jnp.concatenate(
        [ed2, b2, jnp.zeros((NPAD, 14), _f32)], axis=1)

  return pl.pallas_call(
      body,
      out_shape=[
          jax.ShapeDtypeStruct((NPAD, TW2), _f32),
          jax.ShapeDtypeStruct((NPAD, 16), _f32),
      ],
  )(acc1, w2f, a2s, a2d)


def _tc_stage3(acc2):
  def body(acc_ref, out_ref):
    sb = acc_ref[0] + acc_ref[1]
    logits = sb[:, :NC_CLS] / (sb[:, NC_CLS:NC_CLS + 1] + 1e-16)
    mx = jnp.max(logits, axis=1, keepdims=True)
    lse = jnp.log(jnp.sum(jnp.exp(logits - mx), axis=1, keepdims=True)) + mx
    out_ref[...] = (logits - lse)[:N]

  return pl.pallas_call(
      body,
      out_shape=jax.ShapeDtypeStruct((N, NC_CLS), _f32),
  )(acc2)


def _acc_chunks():
  out, off = [], 0
  while off < RPT:
    n = min(BLK, RPT - off)
    out.append((off, n))
    off += n
  return out


def _make_edge_pass(tw, rw, nheads):
  mesh = plsc.VectorSubcoreMesh(core_axis_name="c", subcore_axis_name="s")
  NBUF = 3
  packed = tw != rw

  def body(tsrc, ted, sidx_h, didx_h, out,
           sidx, didx, bufs, tbufs, msgs, acc, *sems):
    sem_gs = sems[0:NBUF]
    sem_gt = sems[NBUF:2 * NBUF]
    sem_sc = sems[2 * NBUF:3 * NBUF]
    c = lax.axis_index("c")
    s = lax.axis_index("s")
    wid = s * 2 + c

    iota = lax.iota(jnp.int32, 16)

    base = wid * NBLK
    pltpu.sync_copy(sidx_h.at[pl.ds(base, NBLK)], sidx)
    pltpu.sync_copy(didx_h.at[pl.ds(base, NBLK)], didx)

    def zrow(i, _):
      for q in range(rw // 16):
        msgs[0, i, pl.ds(q * 16, 16)] = jnp.zeros((16,), _f32)
      return _
    lax.fori_loop(0, BLK, zrow, None)
    row0 = s * RPT
    for off, n in _acc_chunks():
      pltpu.sync_copy(msgs.at[0, pl.ds(0, n)], acc.at[pl.ds(row0 + off, n)])
    plsc.subcore_barrier()

    if nheads == 8:
      def edge_one(b, i):
        t = tbufs[b, i, pl.ds(0, 16)]
        u = bufs[b, i, pl.ds(32, 16)]
        z = u + t
        bb = _perm(t, 8 + (iota & 7))
        ex = jnp.exp(_lrelu(z) - bb)
        msgs[b, i, pl.ds(F1, 16)] = jnp.where(iota < 8, ex, 0.0)
        for g in range(2):
          w = bufs[b, i, pl.ds(g * 16, 16)]
          lo, hi = plsc.unpack(plsc.bitcast(w, jnp.bfloat16),
                               format=plsc.PackFormat.INTERLEAVED,
                               preferred_element_type=_f32)
          e0 = _perm(ex, 4 * g + (iota >> 3))
          e1 = _perm(ex, 4 * g + 2 + (iota >> 3))
          msgs[b, i, pl.ds(32 * g, 16)] = lo * e0
          msgs[b, i, pl.ds(32 * g + 16, 16)] = hi * e1
    else:
      def edge_one(b, i):
        t = tbufs[b, i, pl.ds(0, 16)]
        w1 = bufs[b, i, pl.ds(16, 16)]
        ed = _perm(t, iota * 0)
        bb = _perm(t, iota * 0 + 1)
        es = _perm(w1, iota * 0 + 8)
        ex = jnp.exp(_lrelu(es + ed) - bb)
        w0 = bufs[b, i, pl.ds(0, 16)]
        lo, hi = plsc.unpack(plsc.bitcast(w0, jnp.bfloat16),
                             format=plsc.PackFormat.INTERLEAVED,
                             preferred_element_type=_f32)
        msgs[b, i, pl.ds(0, 16)] = lo * ex
        msgs[b, i, pl.ds(16, 16)] = hi * ex
        w1m = jnp.where(iota == 8, 1.0, w1)
        msgs[b, i, pl.ds(32, 16)] = w1m * ex

    def start_gather(j, b):
      pltpu.make_async_copy(
          tsrc.at[sidx.at[j]], bufs.at[b], sem_gs[b]).start()
      pltpu.make_async_copy(
          ted.at[didx.at[j]], tbufs.at[b], sem_gt[b]).start()

    def wait_gather(j, b):
      pltpu.make_async_copy(
          tsrc.at[sidx.at[j]], bufs.at[b], sem_gs[b]).wait()
      pltpu.make_async_copy(
          ted.at[didx.at[j]], tbufs.at[b], sem_gt[b]).wait()

    def start_scatter(j, b):
      pltpu.async_copy(msgs.at[b], acc.at[didx.at[j]], sem_sc[b], add=True)

    def wait_scatter(j, b):
      pltpu.make_async_copy(
          msgs.at[b], acc.at[didx.at[j]], sem_sc[b]).wait()

    start_gather(0, 0)
    start_gather(1, 1)

    def ring_body(g, _):
      j0 = g * NBUF
      for b in range(NBUF):
        j = j0 + b
        wait_gather(j, b)

        @plsc.parallel_loop(0, BLK, 1, unroll=8)
        def _compute(i, b=b):
          edge_one(b, i)
        start_scatter(j, b)
        bn = (b + 2) % NBUF
        jn = j + 2

        @pl.when(jn < NBLK)
        def _issue():
          @pl.when(j >= 1)
          def _drain():
            wait_scatter(j - 1, bn)
          start_gather(jn, bn)
      return _
    lax.fori_loop(0, NBLK // NBUF, ring_body, None)

    for j in range(NBLK - NBUF, NBLK):
      wait_scatter(j, j % NBUF)
    plsc.subcore_barrier()
    for off, n in _acc_chunks():
      pltpu.sync_copy(acc.at[pl.ds(row0 + off, n)],
                      out.at[c, pl.ds(row0 + off, n)])

  return functools.partial(
      pl.kernel,
      out_type=jax.ShapeDtypeStruct((2, NPAD, rw), _f32),
      mesh=mesh,
      compiler_params=pltpu.CompilerParams(use_tc_tiling_on_sc=False,
                                           needs_layout_passes=False),
      scratch_types=[
          pltpu.VMEM((NBLK, BLK), jnp.int32),
          pltpu.VMEM((NBLK, BLK), jnp.int32),
          pltpu.VMEM((NBUF, BLK, tw), _f32),
          pltpu.VMEM((NBUF, BLK, 16), _f32),
          pltpu.VMEM((NBUF, BLK, rw), _f32),
          pltpu.VMEM_SHARED((NPAD, rw), _f32),
      ] + [pltpu.SemaphoreType.DMA] * (3 * NBUF),
  )(body)


_edge_pass_l1 = _make_edge_pass(TW1, RW1, 8)
_edge_pass_l2 = _make_edge_pass(TW2, RW2, 1)


def kernel(input_matrix, adjacency_coo_matrix, W1, a_src1, a_dst1,
           W2, a_src2, a_dst2):
  xp = jnp.zeros((NPAD, DIN), _f32).at[:N].set(input_matrix)
  w1f = jnp.transpose(W1, (1, 0, 2)).reshape(DIN, F1)
  hh = jnp.repeat(jnp.arange(H1, dtype=jnp.int32), DH1)
  a_s = jnp.zeros((F1, H1), _f32).at[jnp.arange(F1), hh].set(a_src1.reshape(F1))
  a_d = jnp.zeros((F1, H1), _f32).at[jnp.arange(F1), hh].set(a_dst1.reshape(F1))
  w2f = W2.reshape(F1, NC_CLS)
  a2s = a_src2.reshape(NC_CLS, 1)
  a2d = a_dst2.reshape(NC_CLS, 1)

  adj = adjacency_coo_matrix.astype(jnp.int32)
  ar = jnp.arange(N, dtype=jnp.int32)
  fill = N + jnp.arange(EPAD - E, dtype=jnp.int32) % (NPAD - N)
  src = jnp.concatenate([adj[0], ar, fill]).reshape(EPAD // BLK, BLK)
  dst = jnp.concatenate([adj[1], ar, fill]).reshape(EPAD // BLK, BLK)

  tsrc1, ted1 = _tc_stage1(xp, w1f, a_s, a_d)
  acc1 = _edge_pass_l1(tsrc1, ted1, src, dst)
  tsrc2, ted2 = _tc_stage2(acc1, w2f, a2s, a2d)
  acc2 = _edge_pass_l2(tsrc2, ted2, src, dst)
  return _tc_stage3(acc2)
